# Initial kernel scaffold; baseline (speedup 1.0000x reference)
import jax
import jax.numpy as jnp
from jax.experimental import pallas as pl


def kernel(x, angles, pairs_i, pairs_j):
    raise NotImplementedError("write your pallas kernel here")


# speedup vs baseline: 957.7515x; 957.7515x over previous
import jax
import jax.numpy as jnp
from jax.experimental import pallas as pl
from jax.experimental.pallas import tpu as pltpu

DIM_ = 1024
ROW_BLOCK = 512


def _build_m_kernel(pairs_i_ref, pairs_j_ref, angles_ref, m_ref, cs_ref):
    a = angles_ref[...]
    cs_ref[:, 0:1] = jnp.cos(a)
    cs_ref[:, 1:2] = jnp.sin(a)

    row_ids = jax.lax.broadcasted_iota(jnp.int32, (DIM_, DIM_), 0)
    col_ids = jax.lax.broadcasted_iota(jnp.int32, (DIM_, DIM_), 1)
    m_ref[...] = jnp.where(row_ids == col_ids, 1.0, 0.0).astype(jnp.float32)

    n_steps = pairs_i_ref.shape[0]

    def body(t, _):
        i = pairs_i_ref[t]
        j = pairs_j_ref[t]
        tm = jax.lax.rem(t, DIM_)
        c = cs_ref[pl.ds(tm, 1), 0:1]
        s = cs_ref[pl.ds(tm, 1), 1:2]
        mi = m_ref[pl.ds(i, 1), :]
        mj = m_ref[pl.ds(j, 1), :]
        m_ref[pl.ds(i, 1), :] = c * mi - s * mj
        m_ref[pl.ds(j, 1), :] = s * mi + c * mj
        return 0

    jax.lax.fori_loop(0, n_steps, body, 0)


def _matmul_kernel(x_ref, m_ref, o_ref):
    o_ref[...] = jax.lax.dot_general(
        x_ref[...], m_ref[...],
        dimension_numbers=(((1,), (1,)), ((), ())),
        preferred_element_type=jnp.float32,
    )


def kernel(x, angles, pairs_i, pairs_j):
    dim = angles.shape[0]
    x2d = x.reshape(-1, dim).astype(jnp.float32)
    n_rows = x2d.shape[0]

    m = pl.pallas_call(
        _build_m_kernel,
        out_shape=jax.ShapeDtypeStruct((dim, dim), jnp.float32),
        in_specs=[
            pl.BlockSpec(memory_space=pltpu.SMEM),
            pl.BlockSpec(memory_space=pltpu.SMEM),
            pl.BlockSpec(memory_space=pltpu.VMEM),
        ],
        out_specs=pl.BlockSpec(memory_space=pltpu.VMEM),
        scratch_shapes=[pltpu.VMEM((dim, 2), jnp.float32)],
    )(pairs_i, pairs_j, angles.reshape(dim, 1).astype(jnp.float32))

    grid = (n_rows // ROW_BLOCK,)
    y2d = pl.pallas_call(
        _matmul_kernel,
        out_shape=jax.ShapeDtypeStruct((n_rows, dim), jnp.float32),
        grid=grid,
        in_specs=[
            pl.BlockSpec((ROW_BLOCK, dim), lambda r: (r, 0)),
            pl.BlockSpec((dim, dim), lambda r: (0, 0)),
        ],
        out_specs=pl.BlockSpec((ROW_BLOCK, dim), lambda r: (r, 0)),
    )(x2d, m)

    return y2d.reshape(x.shape).astype(x.dtype)


# speedup vs baseline: 2884.5340x; 3.0118x over previous
import jax
import jax.numpy as jnp
from jax.experimental import pallas as pl
from jax.experimental.pallas import tpu as pltpu

DIM_ = 1024
ROW_BLOCK = 512


def _build_m_kernel(pairs_i_ref, pairs_j_ref, angles_ref, m_ref, cs_ref):
    a = angles_ref[...]
    cs_ref[:, 0:1] = jnp.cos(a)
    cs_ref[:, 1:2] = jnp.sin(a)

    p_ids = jax.lax.broadcasted_iota(jnp.int32, (DIM_ * 8, 128), 0)
    l_ids = jax.lax.broadcasted_iota(jnp.int32, (DIM_ * 8, 128), 1)
    logical_col = 128 * (p_ids % 8) + l_ids
    m_ref[...] = jnp.where(logical_col == p_ids // 8, 1.0, 0.0).astype(
        jnp.float32
    )

    n_steps = pairs_i_ref.shape[0]

    def body(t, _):
        ib = pairs_i_ref[t] * 8
        jb = pairs_j_ref[t] * 8
        tm = jax.lax.rem(t, DIM_)
        c = cs_ref[pl.ds(tm, 1), 0:1]
        s = cs_ref[pl.ds(tm, 1), 1:2]
        mi = m_ref[pl.ds(ib, 8), :]
        mj = m_ref[pl.ds(jb, 8), :]
        m_ref[pl.ds(ib, 8), :] = c * mi - s * mj
        m_ref[pl.ds(jb, 8), :] = s * mi + c * mj
        return 0

    jax.lax.fori_loop(0, n_steps, body, 0, unroll=8)


def _matmul_kernel(x_ref, m_ref, o_ref):
    o_ref[...] = jax.lax.dot_general(
        x_ref[...], m_ref[...],
        dimension_numbers=(((1,), (1,)), ((), ())),
        preferred_element_type=jnp.float32,
    )


def kernel(x, angles, pairs_i, pairs_j):
    dim = angles.shape[0]
    x2d = x.reshape(-1, dim).astype(jnp.float32)
    n_rows = x2d.shape[0]

    m8 = pl.pallas_call(
        _build_m_kernel,
        out_shape=jax.ShapeDtypeStruct((dim * 8, 128), jnp.float32),
        in_specs=[
            pl.BlockSpec(memory_space=pltpu.SMEM),
            pl.BlockSpec(memory_space=pltpu.SMEM),
            pl.BlockSpec(memory_space=pltpu.VMEM),
        ],
        out_specs=pl.BlockSpec(memory_space=pltpu.VMEM),
        scratch_shapes=[pltpu.VMEM((dim, 2), jnp.float32)],
    )(pairs_i, pairs_j, angles.reshape(dim, 1).astype(jnp.float32))
    m = m8.reshape(dim, dim)

    grid = (n_rows // ROW_BLOCK,)
    y2d = pl.pallas_call(
        _matmul_kernel,
        out_shape=jax.ShapeDtypeStruct((n_rows, dim), jnp.float32),
        grid=grid,
        in_specs=[
            pl.BlockSpec((ROW_BLOCK, dim), lambda r: (r, 0)),
            pl.BlockSpec((dim, dim), lambda r: (0, 0)),
        ],
        out_specs=pl.BlockSpec((ROW_BLOCK, dim), lambda r: (r, 0)),
    )(x2d, m)

    return y2d.reshape(x.shape).astype(x.dtype)
